# baseline (speedup 1.0000x reference)
import functools
import jax
import jax.numpy as jnp
from jax import lax
from jax.experimental import pallas as pl
from jax.experimental.pallas import tpu as pltpu
from jax.experimental.pallas import tpu_sc as plsc

_CODEBOOK = 1024
_D = 64
_COMMIT = 0.25


def _argmin_body(z_ref, emb_ref, idx_ref, loss_ref):
    b = pl.program_id(0)
    zb = z_ref[0]
    emb = emb_ref[...]
    zsq = jnp.sum(zb * zb, axis=0)
    esq = jnp.sum(emb * emb, axis=1)
    scores = jax.lax.dot_general(
        zb, emb, (((0,), (1,)), ((), ())),
        preferred_element_type=jnp.float32)
    dist = (zsq[:, None] - 2.0 * scores) + esq[None, :]
    m = jnp.min(dist, axis=1, keepdims=True)
    c_iota = jax.lax.broadcasted_iota(jnp.int32, dist.shape, 1)
    idx = jnp.min(jnp.where(dist == m, c_iota, _CODEBOOK), axis=1)
    idx_ref[0, 0, :] = idx
    part = jnp.sum(m)

    @pl.when(b == 0)
    def _init():
        loss_ref[0, 0] = jnp.float32(0.0)

    loss_ref[0, 0] += part


_NC = 2
_NS = 16
_NW = _NC * _NS
_CHUNK = 128


def _make_gather(n_rows):
    per_w = n_rows // _NW
    n_ch = per_w // _CHUNK
    mesh = plsc.VectorSubcoreMesh(core_axis_name="c", subcore_axis_name="s")

    @functools.partial(
        pl.kernel, mesh=mesh,
        compiler_params=pltpu.CompilerParams(use_tc_tiling_on_sc=False),
        out_type=jax.ShapeDtypeStruct((n_rows, _D), jnp.float32),
        scratch_types=[
            pltpu.VMEM((n_ch, _CHUNK), jnp.int32),
            pltpu.VMEM((per_w, _D), jnp.float32),
            pltpu.SemaphoreType.DMA,
        ],
    )
    def gather_k(table_hbm, idx_hbm, out_hbm, idx_v, rows_v, sem):
        wid = lax.axis_index("s") * _NC + lax.axis_index("c")
        base = wid * per_w
        pltpu.sync_copy(idx_hbm.at[wid], idx_v)
        copies = []
        for j in range(n_ch):
            copies.append(pltpu.async_copy(
                table_hbm.at[idx_v.at[j]],
                rows_v.at[pl.ds(j * _CHUNK, _CHUNK)], sem))
        for c in copies:
            c.wait()
        pltpu.sync_copy(rows_v, out_hbm.at[pl.ds(base, per_w)])

    return gather_k


def kernel(z, embedding):
    B, D, H, W = z.shape
    P = H * W
    N = B * P
    z3 = z.reshape(B, D, P)

    idx3, loss_raw = pl.pallas_call(
        _argmin_body,
        grid=(B,),
        in_specs=[
            pl.BlockSpec((1, D, P), lambda b: (b, 0, 0)),
            pl.BlockSpec((_CODEBOOK, D), lambda b: (0, 0)),
        ],
        out_specs=[
            pl.BlockSpec((1, 1, P), lambda b: (b, 0, 0)),
            pl.BlockSpec((1, 1), lambda b: (0, 0),
                         memory_space=pltpu.SMEM),
        ],
        out_shape=[
            jax.ShapeDtypeStruct((B, 1, P), jnp.int32),
            jax.ShapeDtypeStruct((1, 1), jnp.float32),
        ],
    )(z3, embedding)

    idx_w = idx3.reshape(_NW, (N // _NW) // _CHUNK, _CHUNK)
    zq_flat = _make_gather(N)(embedding, idx_w)

    z_q = jnp.transpose(zq_flat.reshape(B, H, W, D), (0, 3, 1, 2))
    indices = idx3.reshape(B, H, W)
    loss = loss_raw[0, 0] * (_COMMIT / (N * D))
    return (z_q, loss, indices)
